# Initial kernel scaffold; baseline (speedup 1.0000x reference)
import jax
import jax.numpy as jnp
from jax.experimental import pallas as pl


def kernel(rel_pos, embeddings):
    raise NotImplementedError("write your pallas kernel here")


# speedup vs baseline: 4.0772x; 4.0772x over previous
import functools

import jax
import jax.numpy as jnp
from jax import lax
from jax.experimental import pallas as pl
from jax.experimental.pallas import tpu as pltpu
from jax.experimental.pallas import tpu_sc as plsc

D_MODEL = 64
MAX_REL = 1024
N_TOTAL = 2048 * 2048

_NC = 2
_NS = 16
_NW = _NC * _NS
_PER_W = N_TOTAL // _NW
_C = 128
_NCHUNK = _PER_W // _C
_L = 16


def _body(rel_hbm, emb_hbm, out_hbm, idx_raw, idx_loc, rows, rows64, sem):
    wid = lax.axis_index("s") * _NC + lax.axis_index("c")
    wbase = wid * _PER_W

    def chunk(c, carry):
        base = wbase + c * _C
        pltpu.sync_copy(rel_hbm.at[pl.ds(base, _C)], idx_raw)
        for i in range(_C // _L):
            v = idx_raw[pl.ds(i * _L, _L)]
            v = jnp.minimum(jnp.maximum(v, -MAX_REL), MAX_REL) + MAX_REL
            idx_loc[pl.ds(i * _L, _L)] = v
        pltpu.async_copy(emb_hbm.at[idx_loc], rows, sem).wait()

        def compact(j, carry2):
            for q in range(D_MODEL // _L):
                rows64[j, pl.ds(q * _L, _L)] = rows[j, pl.ds(q * _L, _L)]
            return carry2

        lax.fori_loop(0, _C, compact, 0)
        pltpu.sync_copy(rows64, out_hbm.at[pl.ds(base, _C)])
        return carry

    lax.fori_loop(0, _NCHUNK, chunk, 0)


@jax.jit
def _sc_gather(rel_flat, emb_padded):
    mesh = plsc.VectorSubcoreMesh(core_axis_name="c", subcore_axis_name="s")
    return pl.kernel(
        _body,
        out_type=jax.ShapeDtypeStruct((N_TOTAL, D_MODEL), jnp.float32),
        mesh=mesh,
        scratch_types=[
            pltpu.VMEM((_C,), jnp.int32),
            pltpu.VMEM((_C,), jnp.int32),
            pltpu.VMEM((_C, 2 * D_MODEL), jnp.float32),
            pltpu.VMEM((_C, D_MODEL), jnp.float32),
            pltpu.SemaphoreType.DMA,
        ],
        compiler_params=pltpu.CompilerParams(use_tc_tiling_on_sc=True),
    )(rel_flat, emb_padded)


def kernel(rel_pos, embeddings):
    s0, s1 = rel_pos.shape
    emb_padded = jnp.pad(embeddings, ((0, 0), (0, D_MODEL)))
    out = _sc_gather(rel_pos.reshape(-1), emb_padded)
    return out.reshape(s0, s1, D_MODEL)


# speedup vs baseline: 4.7766x; 1.1715x over previous
import jax
import jax.numpy as jnp
from jax import lax
from jax.experimental import pallas as pl
from jax.experimental.pallas import tpu as pltpu
from jax.experimental.pallas import tpu_sc as plsc

D_MODEL = 64
MAX_REL = 1024
N_TOTAL = 2048 * 2048

_NC = 2
_NS = 16
_NW = _NC * _NS
_PER_W = N_TOTAL // _NW
_C = 128
_NCHUNK = _PER_W // _C
_L = 16


def _body(rel_hbm, emb_hbm, out_hbm,
          idx_raw, idx_loc, rows, rows64,
          sem_i0, sem_i1, sem_g0, sem_g1, sem_o0, sem_o1):
    wid = lax.axis_index("s") * _NC + lax.axis_index("c")
    wbase = wid * _PER_W
    sem_i = (sem_i0, sem_i1)
    sem_g = (sem_g0, sem_g1)
    sem_o = (sem_o0, sem_o1)

    def start_idx(c, p):
        pltpu.async_copy(rel_hbm.at[pl.ds(wbase + c * _C, _C)],
                         idx_raw.at[p], sem_i[p])

    def wait_idx(p):
        pltpu.make_async_copy(rel_hbm.at[pl.ds(wbase, _C)],
                              idx_raw.at[p], sem_i[p]).wait()

    def clamp(p):
        for i in range(_C // _L):
            v = idx_raw[p, pl.ds(i * _L, _L)]
            v = jnp.minimum(jnp.maximum(v, -MAX_REL), MAX_REL) + MAX_REL
            idx_loc[p, pl.ds(i * _L, _L)] = v

    def start_gather(p):
        pltpu.async_copy(emb_hbm.at[idx_loc.at[p]], rows.at[p], sem_g[p])

    def wait_gather(p):
        pltpu.make_async_copy(emb_hbm.at[idx_loc.at[p]], rows.at[p],
                              sem_g[p]).wait()

    def compact(p):
        def rowcopy(j, carry):
            for r in range(4):
                for q in range(D_MODEL // _L):
                    rows64[p, j * 4 + r, pl.ds(q * _L, _L)] = (
                        rows[p, j * 4 + r, pl.ds(q * _L, _L)])
            return carry
        lax.fori_loop(0, _C // 4, rowcopy, 0)

    def start_out(c, p):
        pltpu.async_copy(rows64.at[p],
                         out_hbm.at[pl.ds(wbase + c * _C, _C)], sem_o[p])

    def wait_out(p):
        pltpu.make_async_copy(rows64.at[p],
                              out_hbm.at[pl.ds(wbase, _C)], sem_o[p]).wait()

    start_idx(0, 0)
    wait_idx(0)
    clamp(0)
    start_gather(0)
    start_idx(1, 1)

    def half(k, c, p):
        q = 1 - p
        wait_idx(q)
        clamp(q)
        start_gather(q)
        @pl.when(c + 2 < _NCHUNK)
        def _():
            start_idx(c + 2, p)
        wait_gather(p)
        @pl.when(c >= 2)
        def _():
            wait_out(p)
        compact(p)
        start_out(c, p)

    def pair(k, carry):
        half(k, 2 * k, 0)
        half(k, 2 * k + 1, 1)
        return carry

    lax.fori_loop(0, _NCHUNK // 2 - 1, pair, 0)

    wait_idx(1)
    clamp(1)
    start_gather(1)
    wait_gather(0)
    wait_out(0)
    compact(0)
    start_out(_NCHUNK - 2, 0)
    wait_gather(1)
    wait_out(1)
    compact(1)
    start_out(_NCHUNK - 1, 1)
    wait_out(0)
    wait_out(1)


@jax.jit
def _sc_gather(rel_flat, emb_padded):
    mesh = plsc.VectorSubcoreMesh(core_axis_name="c", subcore_axis_name="s")
    return pl.kernel(
        _body,
        out_type=jax.ShapeDtypeStruct((N_TOTAL, D_MODEL), jnp.float32),
        mesh=mesh,
        scratch_types=[
            pltpu.VMEM((2, _C), jnp.int32),
            pltpu.VMEM((2, _C), jnp.int32),
            pltpu.VMEM((2, _C, 2 * D_MODEL), jnp.float32),
            pltpu.VMEM((2, _C, D_MODEL), jnp.float32),
            pltpu.SemaphoreType.DMA,
            pltpu.SemaphoreType.DMA,
            pltpu.SemaphoreType.DMA,
            pltpu.SemaphoreType.DMA,
            pltpu.SemaphoreType.DMA,
            pltpu.SemaphoreType.DMA,
        ],
        compiler_params=pltpu.CompilerParams(use_tc_tiling_on_sc=True),
    )(rel_flat, emb_padded)


def kernel(rel_pos, embeddings):
    s0, s1 = rel_pos.shape
    emb_padded = jnp.pad(embeddings, ((0, 0), (0, D_MODEL)))
    out = _sc_gather(rel_pos.reshape(-1), emb_padded)
    return out.reshape(s0, s1, D_MODEL)
